# baseline (speedup 1.0000x reference)
import functools

import jax
import jax.numpy as jnp
from jax import lax
from jax.experimental import pallas as pl
from jax.experimental.pallas import tpu as pltpu
from jax.experimental.pallas import tpu_sc as plsc

NC, NS, L = 2, 16, 16
NW = NC * NS

N0, N1, N2, C = 100000, 25000, 6250, 128
BLK = 40


def _cdiv(a, b):
    return (a + b - 1) // b


def _seg_geometry(nseg):
    nseg_pad = _cdiv(nseg, 8) * 8
    H = _cdiv(nseg // 2, 128) * 128
    own = (H, nseg - H)
    wrote = (H, nseg_pad - H)
    accr = _cdiv(max(own) + 8 + NS * BLK, 8) * 8
    return nseg_pad, H, own, wrote, accr


_PAD1, _H1, _OWN1, _WROTE1, _ACCR1 = _seg_geometry(N1)
_PAD2, _H2, _OWN2, _WROTE2, _ACCR2 = _seg_geometry(N2)
_ACCR = max(_ACCR1, _ACCR2)


def _chunks(total, parts, unit=8):
    u = total // unit
    base, extra = divmod(u, parts)
    out, off = [], 0
    for j in range(parts):
        sz = (base + (1 if j < extra else 0)) * unit
        out.append((off, sz))
        off += sz
    assert off == total
    return out


def _mm_body(x_ref, w_ref, o_ref):
    o_ref[...] = lax.dot_general(
        x_ref[...], w_ref[...], (((1,), (1,)), ((), ())),
        preferred_element_type=jnp.float32)


def _matmul_t(x, w, bm):
    m, c = x.shape
    return pl.pallas_call(
        _mm_body,
        grid=(_cdiv(m, bm),),
        in_specs=[pl.BlockSpec((bm, c), lambda i: (i, 0)),
                  pl.BlockSpec((c, c), lambda i: (0, 0))],
        out_specs=pl.BlockSpec((bm, c), lambda i: (i, 0)),
        out_shape=jax.ShapeDtypeStruct((m, c), jnp.float32),
    )(x, w)


def _mean_mm_body(base_ref, s_ref, c_ref, w_ref, o_ref):
    cnt = jnp.maximum(c_ref[...][:, 0:1], 1.0)
    mean = s_ref[...] / cnt
    o_ref[...] = base_ref[...] + lax.dot_general(
        mean, w_ref[...], (((1,), (1,)), ((), ())),
        preferred_element_type=jnp.float32)


def _mean_mm_add(base, sums, cnts, w, bm):
    m, c = base.shape
    return pl.pallas_call(
        _mean_mm_body,
        grid=(_cdiv(m, bm),),
        in_specs=[pl.BlockSpec((bm, c), lambda i: (i, 0)),
                  pl.BlockSpec((bm, c), lambda i: (i, 0)),
                  pl.BlockSpec((bm, c), lambda i: (i, 0)),
                  pl.BlockSpec((c, c), lambda i: (0, 0))],
        out_specs=pl.BlockSpec((bm, c), lambda i: (i, 0)),
        out_shape=jax.ShapeDtypeStruct((m, c), jnp.float32),
    )(base, sums, cnts, w)


def _sc_fused(x0, cluster1, x1, cluster2, t1, t2):
    zchunks = _chunks(_ACCR, NS)
    zmax = max(sz for _, sz in zchunks)
    zeros = jnp.zeros((zmax, C), jnp.float32)
    ones = jnp.ones((BLK, C), jnp.float32)

    @functools.partial(
        pl.kernel,
        out_type=(jax.ShapeDtypeStruct((N0, C), jnp.float32),
                  jax.ShapeDtypeStruct((N1, C), jnp.float32),
                  jax.ShapeDtypeStruct((_PAD1, C), jnp.float32),
                  jax.ShapeDtypeStruct((_PAD1, C), jnp.float32),
                  jax.ShapeDtypeStruct((_PAD2, C), jnp.float32),
                  jax.ShapeDtypeStruct((_PAD2, C), jnp.float32)),
        mesh=plsc.VectorSubcoreMesh(core_axis_name="c", subcore_axis_name="s",
                                    num_cores=NC, num_subcores=NS),
        scratch_types=[
            pltpu.VMEM((BLK,), jnp.int32),
            pltpu.VMEM((BLK,), jnp.int32),
            pltpu.VMEM((BLK,), jnp.int32),
            pltpu.VMEM((BLK, C), jnp.float32),
            pltpu.VMEM_SHARED((_ACCR, C), jnp.float32),
            pltpu.SemaphoreType.DMA,
        ],
    )
    def k(x0_hbm, cl1_hbm, x1_hbm, cl2_hbm, t1_hbm, t2_hbm,
          z_hbm, o_hbm,
          y0_hbm, y1p_hbm, s1_hbm, c1_hbm, s2_hbm, c2_hbm,
          iv, iiv, liv, xv, sacc, sem):
        cid = lax.axis_index("c")
        sid = lax.axis_index("s")
        wid = sid * NC + cid

        @pl.loop(0, BLK, step=L)
        def _(j):
            iiv[pl.ds(j, L)] = lax.iota(jnp.int32, L) + (sid * BLK + j)

        mine = pl.ds(sid * BLK, BLK)

        def gather_phase(x_hbm, cl_hbm, tab_hbm, y_hbm, n):
            @pl.loop(wid, n // BLK, step=NW)
            def _(b):
                rows = pl.ds(b * BLK, BLK)
                pltpu.sync_copy(cl_hbm.at[rows], iv)
                pltpu.sync_copy(x_hbm.at[rows], sacc.at[mine])
                pltpu.async_copy(tab_hbm.at[iv], xv, sem).wait()
                pltpu.sync_copy(xv, sacc.at[iiv], add=True)
                pltpu.sync_copy(sacc.at[mine], y_hbm.at[rows])

        gather_phase(x0_hbm, cl1_hbm, t1_hbm, y0_hbm, N0)
        gather_phase(x1_hbm, cl2_hbm, t2_hbm, y1p_hbm, N1)

        def zero_acc():
            plsc.subcore_barrier()
            for ss, (off, sz) in enumerate(zchunks):
                if sz == 0:
                    continue

                @pl.when(sid == ss)
                def _(off=off, sz=sz):
                    pltpu.sync_copy(z_hbm.at[pl.ds(0, sz)],
                                    sacc.at[pl.ds(off, sz)])
            plsc.subcore_barrier()

        def writeback(out_hbm, wrote, H):
            plsc.subcore_barrier()
            for cc in range(NC):
                for ss, (off, sz) in enumerate(_chunks(wrote[cc], NS)):
                    if sz == 0:
                        continue

                    @pl.when((cid == cc) & (sid == ss))
                    def _(off=off, sz=sz, ob=cc * H):
                        pltpu.sync_copy(sacc.at[pl.ds(off, sz)],
                                        out_hbm.at[pl.ds(ob + off, sz)])

        def segsum_phase(x_hbm, cl_hbm, sums_hbm, cnts_hbm, n,
                         H, own, wrote):
            base = cid * H
            nown = jnp.where(cid == 0, own[0], own[1])

            def scatter_sweep(load_x):
                @pl.loop(sid, n // BLK, step=NS)
                def _(b):
                    rows = pl.ds(b * BLK, BLK)
                    pltpu.sync_copy(cl_hbm.at[rows], iv)
                    if load_x:
                        pltpu.sync_copy(x_hbm.at[rows], xv)

                    @pl.loop(0, BLK, step=L)
                    def _(j):
                        v = iv[pl.ds(j, L)] - base
                        inb = (v >= 0) & (v < nown)
                        trash = nown + 8 + iiv[pl.ds(j, L)]
                        liv[pl.ds(j, L)] = jnp.where(inb, v, trash)

                    pltpu.sync_copy(xv, sacc.at[liv], add=True)

            zero_acc()
            scatter_sweep(True)
            writeback(sums_hbm, wrote, H)

            zero_acc()
            pltpu.sync_copy(o_hbm.at[pl.ds(0, BLK)], xv)
            scatter_sweep(False)
            writeback(cnts_hbm, wrote, H)

        segsum_phase(x0_hbm, cl1_hbm, s1_hbm, c1_hbm, N0, _H1, _OWN1, _WROTE1)
        segsum_phase(x1_hbm, cl2_hbm, s2_hbm, c2_hbm, N1, _H2, _OWN2, _WROTE2)

    return k(x0, cluster1, x1, cluster2, t1, t2, zeros, ones)


def kernel(x0, x1, x2, cluster1, cluster2, Wf0, Wf1, Wc0, Wc1):
    t1 = _matmul_t(x1, Wc0, bm=1000)
    t2 = _matmul_t(x2, Wc1, bm=512)

    y0, y1p, s1, c1, s2, c2 = _sc_fused(x0, cluster1, x1, cluster2, t1, t2)

    y1 = _mean_mm_add(y1p, s1, c1, Wf0, bm=1000)
    y2 = _mean_mm_add(x2, s2, c2, Wf1, bm=512)
    return (y0, y1, y2)


# speedup vs baseline: 1.4607x; 1.4607x over previous
import functools

import jax
import jax.numpy as jnp
from jax import lax
from jax.experimental import pallas as pl
from jax.experimental.pallas import tpu as pltpu
from jax.experimental.pallas import tpu_sc as plsc

NC, NS, L = 2, 16, 16
NW = NC * NS

N0, N1, N2, C = 100000, 25000, 6250, 128
G0 = 160
G1R = 40
TRASH = 16


def _cdiv(a, b):
    return (a + b - 1) // b


def _seg_geometry(nseg):
    nseg_pad = _cdiv(nseg, 8) * 8
    H = _cdiv(nseg // 2, 128) * 128
    own = (H, nseg - H)
    wrote = (H, nseg_pad - H)
    accr = _cdiv(max(own) + 8 + TRASH, 8) * 8
    return nseg_pad, H, own, wrote, accr


_PAD1, _H1, _OWN1, _WROTE1, _ACCR1 = _seg_geometry(N1)
_PAD2, _H2, _OWN2, _WROTE2, _ACCR2 = _seg_geometry(N2)
_ACCR = max(_ACCR1, _ACCR2)


def _chunks(total, parts, unit=8):
    u = total // unit
    base, extra = divmod(u, parts)
    out, off = [], 0
    for j in range(parts):
        sz = (base + (1 if j < extra else 0)) * unit
        out.append((off, sz))
        off += sz
    assert off == total
    return out


def _mm_body(x_ref, w_ref, o_ref):
    o_ref[...] = lax.dot_general(
        x_ref[...], w_ref[...], (((1,), (1,)), ((), ())),
        preferred_element_type=jnp.float32)


def _matmul_t(x, w, bm):
    m, c = x.shape
    return pl.pallas_call(
        _mm_body,
        grid=(_cdiv(m, bm),),
        in_specs=[pl.BlockSpec((bm, c), lambda i: (i, 0)),
                  pl.BlockSpec((c, c), lambda i: (0, 0))],
        out_specs=pl.BlockSpec((bm, c), lambda i: (i, 0)),
        out_shape=jax.ShapeDtypeStruct((m, c), jnp.float32),
    )(x, w)


def _add_body(a_ref, b_ref, o_ref):
    o_ref[...] = a_ref[...] + b_ref[...]


def _add(a, b, bm):
    m, c = a.shape
    return pl.pallas_call(
        _add_body,
        grid=(_cdiv(m, bm),),
        in_specs=[pl.BlockSpec((bm, c), lambda i: (i, 0)),
                  pl.BlockSpec((bm, c), lambda i: (i, 0))],
        out_specs=pl.BlockSpec((bm, c), lambda i: (i, 0)),
        out_shape=jax.ShapeDtypeStruct((m, c), jnp.float32),
    )(a, b)


def _mean_mm_body(base_ref, b2_ref, s_ref, c_ref, w_ref, o_ref):
    cnt = jnp.maximum(c_ref[...][:, 0:1], 1.0)
    mean = s_ref[...] / cnt
    o_ref[...] = base_ref[...] + b2_ref[...] + lax.dot_general(
        mean, w_ref[...], (((1,), (1,)), ((), ())),
        preferred_element_type=jnp.float32)


def _mean_mm_add(base, base2, sums, cnts, w, bm):
    m, c = base.shape
    return pl.pallas_call(
        _mean_mm_body,
        grid=(_cdiv(m, bm),),
        in_specs=[pl.BlockSpec((bm, c), lambda i: (i, 0)),
                  pl.BlockSpec((bm, c), lambda i: (i, 0)),
                  pl.BlockSpec((bm, c), lambda i: (i, 0)),
                  pl.BlockSpec((bm, c), lambda i: (i, 0)),
                  pl.BlockSpec((c, c), lambda i: (0, 0))],
        out_specs=pl.BlockSpec((bm, c), lambda i: (i, 0)),
        out_shape=jax.ShapeDtypeStruct((m, c), jnp.float32),
    )(base, base2, sums, cnts, w)


def _sc_fused(x0, cluster1, x1, cluster2, t1, t2):
    zchunks = _chunks(_ACCR, NS)
    zmax = max(sz for _, sz in zchunks)
    zeros = jnp.zeros((zmax, C), jnp.float32)
    ones = jnp.ones((G0, C), jnp.float32)

    @functools.partial(
        pl.kernel,
        out_type=(jax.ShapeDtypeStruct((N0, C), jnp.float32),
                  jax.ShapeDtypeStruct((N1, C), jnp.float32),
                  jax.ShapeDtypeStruct((_PAD1, C), jnp.float32),
                  jax.ShapeDtypeStruct((_PAD1, C), jnp.float32),
                  jax.ShapeDtypeStruct((_PAD2, C), jnp.float32),
                  jax.ShapeDtypeStruct((_PAD2, C), jnp.float32)),
        mesh=plsc.VectorSubcoreMesh(core_axis_name="c", subcore_axis_name="s",
                                    num_cores=NC, num_subcores=NS),
        scratch_types=[
            pltpu.VMEM((G0,), jnp.int32),
            pltpu.VMEM((128,), jnp.int32),
            pltpu.VMEM((32,), jnp.int32),
            pltpu.VMEM((G1R,), jnp.int32),
            pltpu.VMEM((G0, C), jnp.float32),
            pltpu.VMEM_SHARED((_ACCR, C), jnp.float32),
            pltpu.SemaphoreType.DMA,
        ],
    )
    def k(x0_hbm, cl1_hbm, x1_hbm, cl2_hbm, t1_hbm, t2_hbm,
          z_hbm, o_hbm,
          g1_hbm, g2_hbm, s1_hbm, c1_hbm, s2_hbm, c2_hbm,
          iv, liv128, liv32, liv40, xv, sacc, sem):
        cid = lax.axis_index("c")
        sid = lax.axis_index("s")
        wid = sid * NC + cid

        def gather_phase(cl_hbm, tab_hbm, g_hbm, n, grp, splits):
            @pl.loop(wid, n // grp, step=NW)
            def _(b):
                rows = pl.ds(b * grp, grp)
                pltpu.sync_copy(cl_hbm.at[rows], iv.at[pl.ds(0, grp)])
                off = 0
                for sz in splits:
                    pltpu.async_copy(tab_hbm.at[iv.at[pl.ds(off, sz)]],
                                     xv.at[pl.ds(off, sz)], sem).wait()
                    off += sz
                pltpu.sync_copy(xv.at[pl.ds(0, grp)], g_hbm.at[rows])

        gather_phase(cl1_hbm, t1_hbm, g1_hbm, N0, G0, (128, 32))
        gather_phase(cl2_hbm, t2_hbm, g2_hbm, N1, G1R, (G1R,))

        def zero_acc():
            plsc.subcore_barrier()
            for ss, (off, sz) in enumerate(zchunks):
                if sz == 0:
                    continue

                @pl.when(sid == ss)
                def _(off=off, sz=sz):
                    pltpu.sync_copy(z_hbm.at[pl.ds(0, sz)],
                                    sacc.at[pl.ds(off, sz)])
            plsc.subcore_barrier()

        def writeback(out_hbm, wrote, H):
            plsc.subcore_barrier()
            for cc in range(NC):
                for ss, (off, sz) in enumerate(_chunks(wrote[cc], NS)):
                    if sz == 0:
                        continue

                    @pl.when((cid == cc) & (sid == ss))
                    def _(off=off, sz=sz, ob=cc * H):
                        pltpu.sync_copy(sacc.at[pl.ds(off, sz)],
                                        out_hbm.at[pl.ds(ob + off, sz)])

        def segsum_phase(x_hbm, cl_hbm, sums_hbm, cnts_hbm, n,
                         H, own, wrote, grp, parts):
            base = cid * H
            nown = jnp.where(cid == 0, own[0], own[1])

            def scatter_sweep(load_x):
                @pl.loop(sid, n // grp, step=NS)
                def _(b):
                    rows = pl.ds(b * grp, grp)
                    pltpu.sync_copy(cl_hbm.at[rows], iv.at[pl.ds(0, grp)])
                    if load_x:
                        pltpu.sync_copy(x_hbm.at[rows], xv.at[pl.ds(0, grp)])

                    off = 0
                    for lref, sz in parts:
                        @pl.loop(0, sz, step=L)
                        def _(j, off=off, lref=lref):
                            v = iv[pl.ds(off + j, L)] - base
                            inb = (v >= 0) & (v < nown)
                            trash = (nown + 8 +
                                     ((lax.iota(jnp.int32, L) + j) &
                                      (TRASH - 1)))
                            lref[pl.ds(j, L)] = jnp.where(inb, v, trash)
                        off += sz

                    off = 0
                    for lref, sz in parts:
                        pltpu.sync_copy(xv.at[pl.ds(off, sz)],
                                        sacc.at[lref], add=True)
                        off += sz

            zero_acc()
            scatter_sweep(True)
            writeback(sums_hbm, wrote, H)

            zero_acc()
            pltpu.sync_copy(o_hbm.at[pl.ds(0, G0)], xv)
            scatter_sweep(False)
            writeback(cnts_hbm, wrote, H)

        segsum_phase(x0_hbm, cl1_hbm, s1_hbm, c1_hbm, N0,
                     _H1, _OWN1, _WROTE1, G0, ((liv128, 128), (liv32, 32)))
        segsum_phase(x1_hbm, cl2_hbm, s2_hbm, c2_hbm, N1,
                     _H2, _OWN2, _WROTE2, G1R, ((liv40, G1R),))

    return k(x0, cluster1, x1, cluster2, t1, t2, zeros, ones)


def kernel(x0, x1, x2, cluster1, cluster2, Wf0, Wf1, Wc0, Wc1):
    t1 = _matmul_t(x1, Wc0, bm=1000)
    t2 = _matmul_t(x2, Wc1, bm=512)

    g1, g2, s1, c1, s2, c2 = _sc_fused(x0, cluster1, x1, cluster2, t1, t2)

    y0 = _add(x0, g1, bm=1000)
    y1 = _mean_mm_add(x1, g2, s1, c1, Wf0, bm=1000)
    zero2 = jnp.zeros((N2, C), jnp.float32)
    y2 = _mean_mm_add(x2, zero2, s2, c2, Wf1, bm=512)
    return (y0, y1, y2)


# speedup vs baseline: 1.8612x; 1.2742x over previous
import functools

import jax
import jax.numpy as jnp
from jax import lax
from jax.experimental import pallas as pl
from jax.experimental.pallas import tpu as pltpu
from jax.experimental.pallas import tpu_sc as plsc

NC, NS, L = 2, 16, 16
NW = NC * NS

N0, N1, N2, C = 100000, 25000, 6250, 128
G0 = 80
G1R = 40
TRASH = 16


def _cdiv(a, b):
    return (a + b - 1) // b


def _seg_geometry(nseg):
    nseg_pad = _cdiv(nseg, 8) * 8
    H = _cdiv(nseg // 2, 128) * 128
    own = (H, nseg - H)
    wrote = (H, nseg_pad - H)
    accr = _cdiv(max(own) + 8 + TRASH, 8) * 8
    return nseg_pad, H, own, wrote, accr


_PAD1, _H1, _OWN1, _WROTE1, _ACCR1 = _seg_geometry(N1)
_PAD2, _H2, _OWN2, _WROTE2, _ACCR2 = _seg_geometry(N2)
_ACCR = max(_ACCR1, _ACCR2)


def _chunks(total, parts, unit=8):
    u = total // unit
    base, extra = divmod(u, parts)
    out, off = [], 0
    for j in range(parts):
        sz = (base + (1 if j < extra else 0)) * unit
        out.append((off, sz))
        off += sz
    assert off == total
    return out


def _mm_body(x_ref, w_ref, o_ref):
    o_ref[...] = lax.dot_general(
        x_ref[...], w_ref[...], (((1,), (1,)), ((), ())),
        preferred_element_type=jnp.float32)


def _matmul_t(x, w, bm):
    m, c = x.shape
    return pl.pallas_call(
        _mm_body,
        grid=(_cdiv(m, bm),),
        in_specs=[pl.BlockSpec((bm, c), lambda i: (i, 0)),
                  pl.BlockSpec((c, c), lambda i: (0, 0))],
        out_specs=pl.BlockSpec((bm, c), lambda i: (i, 0)),
        out_shape=jax.ShapeDtypeStruct((m, c), jnp.float32),
    )(x, w)


def _add_body(a_ref, b_ref, o_ref):
    o_ref[...] = a_ref[...] + b_ref[...]


def _add(a, b, bm):
    m, c = a.shape
    return pl.pallas_call(
        _add_body,
        grid=(_cdiv(m, bm),),
        in_specs=[pl.BlockSpec((bm, c), lambda i: (i, 0)),
                  pl.BlockSpec((bm, c), lambda i: (i, 0))],
        out_specs=pl.BlockSpec((bm, c), lambda i: (i, 0)),
        out_shape=jax.ShapeDtypeStruct((m, c), jnp.float32),
    )(a, b)


def _mean_mm_body(base_ref, b2_ref, s_ref, c_ref, w_ref, o_ref):
    cnt = jnp.maximum(c_ref[...][:, 0:1], 1.0)
    mean = s_ref[...] / cnt
    o_ref[...] = base_ref[...] + b2_ref[...] + lax.dot_general(
        mean, w_ref[...], (((1,), (1,)), ((), ())),
        preferred_element_type=jnp.float32)


def _mean_mm_add(base, base2, sums, cnts, w, bm):
    m, c = base.shape
    return pl.pallas_call(
        _mean_mm_body,
        grid=(_cdiv(m, bm),),
        in_specs=[pl.BlockSpec((bm, c), lambda i: (i, 0)),
                  pl.BlockSpec((bm, c), lambda i: (i, 0)),
                  pl.BlockSpec((bm, c), lambda i: (i, 0)),
                  pl.BlockSpec((bm, c), lambda i: (i, 0)),
                  pl.BlockSpec((c, c), lambda i: (0, 0))],
        out_specs=pl.BlockSpec((bm, c), lambda i: (i, 0)),
        out_shape=jax.ShapeDtypeStruct((m, c), jnp.float32),
    )(base, base2, sums, cnts, w)


def _sc_fused(x0, cluster1, x1, cluster2, t1, t2):
    zchunks = _chunks(_ACCR, NS)
    zmax = max(sz for _, sz in zchunks)
    zeros = jnp.zeros((zmax, C), jnp.float32)
    ones = jnp.ones((G0, C), jnp.float32)

    @functools.partial(
        pl.kernel,
        out_type=(jax.ShapeDtypeStruct((N0, C), jnp.float32),
                  jax.ShapeDtypeStruct((N1, C), jnp.float32),
                  jax.ShapeDtypeStruct((_PAD1, C), jnp.float32),
                  jax.ShapeDtypeStruct((_PAD1, C), jnp.float32),
                  jax.ShapeDtypeStruct((_PAD2, C), jnp.float32),
                  jax.ShapeDtypeStruct((_PAD2, C), jnp.float32)),
        mesh=plsc.VectorSubcoreMesh(core_axis_name="c", subcore_axis_name="s",
                                    num_cores=NC, num_subcores=NS),
        scratch_types=[
            pltpu.VMEM((G0,), jnp.int32),
            pltpu.VMEM((G0,), jnp.int32),
            pltpu.VMEM((G0,), jnp.int32),
            pltpu.VMEM((G0,), jnp.int32),
            pltpu.VMEM((G1R,), jnp.int32),
            pltpu.VMEM((G1R,), jnp.int32),
            pltpu.VMEM((G0, C), jnp.float32),
            pltpu.VMEM((G0, C), jnp.float32),
            pltpu.VMEM_SHARED((_ACCR, C), jnp.float32),
            pltpu.SemaphoreType.DMA,
            pltpu.SemaphoreType.DMA,
        ],
    )
    def k(x0_hbm, cl1_hbm, x1_hbm, cl2_hbm, t1_hbm, t2_hbm,
          z_hbm, o_hbm,
          g1_hbm, g2_hbm, s1_hbm, c1_hbm, s2_hbm, c2_hbm,
          iva, ivb, liva, livb, liv40a, liv40b, xva, xvb, sacc,
          semA, semB):
        cid = lax.axis_index("c")
        sid = lax.axis_index("s")
        wid = sid * NC + cid

        def pipeline(nblk, first, stride, start, finish):
            cnt = (nblk - 1 - first) // stride + 1

            def blk(i):
                return first + i * stride

            start(blk(0), 0)

            @pl.loop(0, cnt // 2)
            def _(t):
                i0 = 2 * t
                start(blk(i0 + 1), 1)
                finish(blk(i0), 0)

                @pl.when(i0 + 2 < cnt)
                def _():
                    start(blk(i0 + 2), 0)

                finish(blk(i0 + 1), 1)

            @pl.when(cnt % 2 == 1)
            def _():
                finish(blk(cnt - 1), 0)

        bufs = ((iva, xva, semA), (ivb, xvb, semB))

        def gather_phase(cl_hbm, tab_hbm, g_hbm, n, grp):
            def start(b, p):
                iv, xv, sem = bufs[p]
                pltpu.async_copy(cl_hbm.at[pl.ds(b * grp, grp)],
                                 iv.at[pl.ds(0, grp)], sem)

            def finish(b, p):
                iv, xv, sem = bufs[p]
                pltpu.make_async_copy(cl_hbm.at[pl.ds(b * grp, grp)],
                                      iv.at[pl.ds(0, grp)], sem).wait()
                pltpu.async_copy(tab_hbm.at[iv.at[pl.ds(0, grp)]],
                                 xv.at[pl.ds(0, grp)], sem).wait()
                pltpu.sync_copy(xv.at[pl.ds(0, grp)],
                                g_hbm.at[pl.ds(b * grp, grp)])

            pipeline(n // grp, wid, NW, start, finish)

        gather_phase(cl1_hbm, t1_hbm, g1_hbm, N0, G0)
        gather_phase(cl2_hbm, t2_hbm, g2_hbm, N1, G1R)

        def zero_acc():
            plsc.subcore_barrier()
            for ss, (off, sz) in enumerate(zchunks):
                if sz == 0:
                    continue

                @pl.when(sid == ss)
                def _(off=off, sz=sz):
                    pltpu.sync_copy(z_hbm.at[pl.ds(0, sz)],
                                    sacc.at[pl.ds(off, sz)])
            plsc.subcore_barrier()

        def writeback(out_hbm, wrote, H):
            plsc.subcore_barrier()
            for cc in range(NC):
                for ss, (off, sz) in enumerate(_chunks(wrote[cc], NS)):
                    if sz == 0:
                        continue

                    @pl.when((cid == cc) & (sid == ss))
                    def _(off=off, sz=sz, ob=cc * H):
                        pltpu.sync_copy(sacc.at[pl.ds(off, sz)],
                                        out_hbm.at[pl.ds(ob + off, sz)])

        def segsum_phase(x_hbm, cl_hbm, sums_hbm, cnts_hbm, n,
                         H, own, wrote, grp, livs):
            base = cid * H
            nown = jnp.where(cid == 0, own[0], own[1])

            def scatter_sweep(load_x):
                def start(b, p):
                    iv, xv, sem = bufs[p]
                    pltpu.async_copy(cl_hbm.at[pl.ds(b * grp, grp)],
                                     iv.at[pl.ds(0, grp)], sem)
                    if load_x:
                        pltpu.async_copy(x_hbm.at[pl.ds(b * grp, grp)],
                                         xv.at[pl.ds(0, grp)], sem)

                def finish(b, p):
                    iv, xv, sem = bufs[p]
                    lref = livs[p]
                    pltpu.make_async_copy(cl_hbm.at[pl.ds(b * grp, grp)],
                                          iv.at[pl.ds(0, grp)], sem).wait()
                    if load_x:
                        pltpu.make_async_copy(x_hbm.at[pl.ds(b * grp, grp)],
                                              xv.at[pl.ds(0, grp)],
                                              sem).wait()

                    @pl.loop(0, grp, step=L)
                    def _(j):
                        v = iv[pl.ds(j, L)] - base
                        inb = (v >= 0) & (v < nown)
                        trash = (nown + 8 +
                                 ((lax.iota(jnp.int32, L) + j) & (TRASH - 1)))
                        lref[pl.ds(j, L)] = jnp.where(inb, v, trash)

                    pltpu.sync_copy(xv.at[pl.ds(0, grp)],
                                    sacc.at[lref], add=True)

                pipeline(n // grp, sid, NS, start, finish)

            zero_acc()
            scatter_sweep(True)
            writeback(sums_hbm, wrote, H)

            zero_acc()
            pltpu.sync_copy(o_hbm.at[pl.ds(0, G0)], xva)
            pltpu.sync_copy(o_hbm.at[pl.ds(0, G0)], xvb)
            scatter_sweep(False)
            writeback(cnts_hbm, wrote, H)

        segsum_phase(x0_hbm, cl1_hbm, s1_hbm, c1_hbm, N0,
                     _H1, _OWN1, _WROTE1, G0, (liva, livb))
        segsum_phase(x1_hbm, cl2_hbm, s2_hbm, c2_hbm, N1,
                     _H2, _OWN2, _WROTE2, G1R, (liv40a, liv40b))

    return k(x0, cluster1, x1, cluster2, t1, t2, zeros, ones)


def kernel(x0, x1, x2, cluster1, cluster2, Wf0, Wf1, Wc0, Wc1):
    t1 = _matmul_t(x1, Wc0, bm=1000)
    t2 = _matmul_t(x2, Wc1, bm=512)

    g1, g2, s1, c1, s2, c2 = _sc_fused(x0, cluster1, x1, cluster2, t1, t2)

    y0 = _add(x0, g1, bm=1000)
    y1 = _mean_mm_add(x1, g2, s1, c1, Wf0, bm=1000)
    zero2 = jnp.zeros((N2, C), jnp.float32)
    y2 = _mean_mm_add(x2, zero2, s2, c2, Wf1, bm=512)
    return (y0, y1, y2)


# speedup vs baseline: 1.9041x; 1.0230x over previous
import functools

import jax
import jax.numpy as jnp
from jax import lax
from jax.experimental import pallas as pl
from jax.experimental.pallas import tpu as pltpu
from jax.experimental.pallas import tpu_sc as plsc

NC, NS, L = 2, 16, 16
NW = NC * NS

N0, N1, N2, C = 100000, 25000, 6250, 128
G0 = 80
G1R = 40
TRASH = 16


def _cdiv(a, b):
    return (a + b - 1) // b


def _seg_geometry(nseg):
    nseg_pad = _cdiv(nseg, 8) * 8
    H = _cdiv(nseg // 2, 128) * 128
    own = (H, nseg - H)
    wrote = (H, nseg_pad - H)
    accr = _cdiv(max(own) + 8 + TRASH, 8) * 8
    return nseg_pad, H, own, wrote, accr


_PAD1, _H1, _OWN1, _WROTE1, _ACCR1 = _seg_geometry(N1)
_PAD2, _H2, _OWN2, _WROTE2, _ACCR2 = _seg_geometry(N2)
_ACCR = max(_ACCR1, _ACCR2)


def _chunks(total, parts, unit=8):
    u = total // unit
    base, extra = divmod(u, parts)
    out, off = [], 0
    for j in range(parts):
        sz = (base + (1 if j < extra else 0)) * unit
        out.append((off, sz))
        off += sz
    assert off == total
    return out


def _mm_body(x_ref, w_ref, o_ref):
    o_ref[...] = lax.dot_general(
        x_ref[...], w_ref[...], (((1,), (1,)), ((), ())),
        preferred_element_type=jnp.float32)


def _matmul_t(x, w, bm):
    m, c = x.shape
    return pl.pallas_call(
        _mm_body,
        grid=(_cdiv(m, bm),),
        in_specs=[pl.BlockSpec((bm, c), lambda i: (i, 0)),
                  pl.BlockSpec((c, c), lambda i: (0, 0))],
        out_specs=pl.BlockSpec((bm, c), lambda i: (i, 0)),
        out_shape=jax.ShapeDtypeStruct((m, c), jnp.float32),
    )(x, w)


def _add_body(a_ref, b_ref, o_ref):
    o_ref[...] = a_ref[...] + b_ref[...]


def _add(a, b, bm):
    m, c = a.shape
    return pl.pallas_call(
        _add_body,
        grid=(_cdiv(m, bm),),
        in_specs=[pl.BlockSpec((bm, c), lambda i: (i, 0)),
                  pl.BlockSpec((bm, c), lambda i: (i, 0))],
        out_specs=pl.BlockSpec((bm, c), lambda i: (i, 0)),
        out_shape=jax.ShapeDtypeStruct((m, c), jnp.float32),
    )(a, b)


def _mean_mm_body(base_ref, b2_ref, s_ref, c_ref, w_ref, o_ref):
    cnt = jnp.maximum(c_ref[...][:, 0:1], 1.0)
    mean = s_ref[...] / cnt
    o_ref[...] = base_ref[...] + b2_ref[...] + lax.dot_general(
        mean, w_ref[...], (((1,), (1,)), ((), ())),
        preferred_element_type=jnp.float32)


def _mean_mm_add(base, base2, sums, cnts, w, bm):
    m, c = base.shape
    return pl.pallas_call(
        _mean_mm_body,
        grid=(_cdiv(m, bm),),
        in_specs=[pl.BlockSpec((bm, c), lambda i: (i, 0)),
                  pl.BlockSpec((bm, c), lambda i: (i, 0)),
                  pl.BlockSpec((bm, c), lambda i: (i, 0)),
                  pl.BlockSpec((bm, c), lambda i: (i, 0)),
                  pl.BlockSpec((c, c), lambda i: (0, 0))],
        out_specs=pl.BlockSpec((bm, c), lambda i: (i, 0)),
        out_shape=jax.ShapeDtypeStruct((m, c), jnp.float32),
    )(base, base2, sums, cnts, w)


def _sc_fused(x0, cluster1, x1, cluster2, t1, t2):
    zchunks1 = _chunks(_ACCR1, NS)
    zchunks2 = _chunks(_ACCR2, NS)
    zmax = max(sz for _, sz in zchunks1 + zchunks2)
    zeros = jnp.zeros((zmax, C), jnp.float32)
    ones = jnp.ones((G0, C), jnp.float32)

    @functools.partial(
        pl.kernel,
        out_type=(jax.ShapeDtypeStruct((N0, C), jnp.float32),
                  jax.ShapeDtypeStruct((N1, C), jnp.float32),
                  jax.ShapeDtypeStruct((_PAD1, C), jnp.float32),
                  jax.ShapeDtypeStruct((_PAD1, C), jnp.float32),
                  jax.ShapeDtypeStruct((_PAD2, C), jnp.float32),
                  jax.ShapeDtypeStruct((_PAD2, C), jnp.float32)),
        mesh=plsc.VectorSubcoreMesh(core_axis_name="c", subcore_axis_name="s",
                                    num_cores=NC, num_subcores=NS),
        scratch_types=[
            pltpu.VMEM((G0,), jnp.int32),
            pltpu.VMEM((G0,), jnp.int32),
            pltpu.VMEM((G0,), jnp.int32),
            pltpu.VMEM((G0,), jnp.int32),
            pltpu.VMEM((G1R,), jnp.int32),
            pltpu.VMEM((G1R,), jnp.int32),
            pltpu.VMEM((G0, C), jnp.float32),
            pltpu.VMEM((G0, C), jnp.float32),
            pltpu.VMEM_SHARED((_ACCR, C), jnp.float32),
            pltpu.SemaphoreType.DMA,
            pltpu.SemaphoreType.DMA,
        ],
    )
    def k(x0_hbm, cl1_hbm, x1_hbm, cl2_hbm, t1_hbm, t2_hbm,
          z_hbm, o_hbm,
          g1_hbm, g2_hbm, s1_hbm, c1_hbm, s2_hbm, c2_hbm,
          iva, ivb, liva, livb, liv40a, liv40b, xva, xvb, sacc,
          semA, semB):
        cid = lax.axis_index("c")
        sid = lax.axis_index("s")
        wid = sid * NC + cid

        def pipeline(nblk, first, stride, start, finish):
            cnt = (nblk - 1 - first) // stride + 1

            def blk(i):
                return first + i * stride

            start(blk(0), 0)

            @pl.loop(0, cnt // 2)
            def _(t):
                i0 = 2 * t
                start(blk(i0 + 1), 1)
                finish(blk(i0), 0)

                @pl.when(i0 + 2 < cnt)
                def _():
                    start(blk(i0 + 2), 0)

                finish(blk(i0 + 1), 1)

            @pl.when(cnt % 2 == 1)
            def _():
                finish(blk(cnt - 1), 0)

        bufs = ((iva, xva, semA), (ivb, xvb, semB))

        def gather_phase(cl_hbm, tab_hbm, g_hbm, n, grp):
            def start(b, p):
                iv, xv, sem = bufs[p]
                pltpu.async_copy(cl_hbm.at[pl.ds(b * grp, grp)],
                                 iv.at[pl.ds(0, grp)], sem)

            def finish(b, p):
                iv, xv, sem = bufs[p]
                pltpu.make_async_copy(cl_hbm.at[pl.ds(b * grp, grp)],
                                      iv.at[pl.ds(0, grp)], sem).wait()
                pltpu.async_copy(tab_hbm.at[iv.at[pl.ds(0, grp)]],
                                 xv.at[pl.ds(0, grp)], sem).wait()
                pltpu.sync_copy(xv.at[pl.ds(0, grp)],
                                g_hbm.at[pl.ds(b * grp, grp)])

            pipeline(n // grp, wid, NW, start, finish)

        gather_phase(cl1_hbm, t1_hbm, g1_hbm, N0, G0)
        gather_phase(cl2_hbm, t2_hbm, g2_hbm, N1, G1R)

        def zero_acc(zchunks):
            plsc.subcore_barrier()
            for ss, (off, sz) in enumerate(zchunks):
                if sz == 0:
                    continue

                @pl.when(sid == ss)
                def _(off=off, sz=sz):
                    pltpu.sync_copy(z_hbm.at[pl.ds(0, sz)],
                                    sacc.at[pl.ds(off, sz)])
            plsc.subcore_barrier()

        def writeback(out_hbm, wrote, H):
            plsc.subcore_barrier()
            for cc in range(NC):
                for ss, (off, sz) in enumerate(_chunks(wrote[cc], NS)):
                    if sz == 0:
                        continue

                    @pl.when((cid == cc) & (sid == ss))
                    def _(off=off, sz=sz, ob=cc * H):
                        pltpu.sync_copy(sacc.at[pl.ds(off, sz)],
                                        out_hbm.at[pl.ds(ob + off, sz)])

        def segsum_phase(x_hbm, cl_hbm, sums_hbm, cnts_hbm, n,
                         H, own, wrote, grp, livs, zchunks):
            base = cid * H
            nown = jnp.where(cid == 0, own[0], own[1])

            def scatter_sweep(load_x):
                def start(b, p):
                    iv, xv, sem = bufs[p]
                    pltpu.async_copy(cl_hbm.at[pl.ds(b * grp, grp)],
                                     iv.at[pl.ds(0, grp)], sem)
                    if load_x:
                        pltpu.async_copy(x_hbm.at[pl.ds(b * grp, grp)],
                                         xv.at[pl.ds(0, grp)], sem)

                def finish(b, p):
                    iv, xv, sem = bufs[p]
                    lref = livs[p]
                    pltpu.make_async_copy(cl_hbm.at[pl.ds(b * grp, grp)],
                                          iv.at[pl.ds(0, grp)], sem).wait()
                    if load_x:
                        pltpu.make_async_copy(x_hbm.at[pl.ds(b * grp, grp)],
                                              xv.at[pl.ds(0, grp)],
                                              sem).wait()

                    @pl.loop(0, grp, step=L)
                    def _(j):
                        v = iv[pl.ds(j, L)] - base
                        inb = (v >= 0) & (v < nown)
                        trash = (nown + 8 +
                                 ((lax.iota(jnp.int32, L) + j) & (TRASH - 1)))
                        lref[pl.ds(j, L)] = jnp.where(inb, v, trash)

                    pltpu.sync_copy(xv.at[pl.ds(0, grp)],
                                    sacc.at[lref], add=True)

                pipeline(n // grp, sid, NS, start, finish)

            zero_acc(zchunks)
            scatter_sweep(True)
            writeback(sums_hbm, wrote, H)

            zero_acc(zchunks)
            pltpu.sync_copy(o_hbm.at[pl.ds(0, G0)], xva)
            pltpu.sync_copy(o_hbm.at[pl.ds(0, G0)], xvb)
            scatter_sweep(False)
            writeback(cnts_hbm, wrote, H)

        segsum_phase(x0_hbm, cl1_hbm, s1_hbm, c1_hbm, N0,
                     _H1, _OWN1, _WROTE1, G0, (liva, livb), zchunks1)
        segsum_phase(x1_hbm, cl2_hbm, s2_hbm, c2_hbm, N1,
                     _H2, _OWN2, _WROTE2, G1R, (liv40a, liv40b), zchunks2)

    return k(x0, cluster1, x1, cluster2, t1, t2, zeros, ones)


def kernel(x0, x1, x2, cluster1, cluster2, Wf0, Wf1, Wc0, Wc1):
    t1 = _matmul_t(x1, Wc0, bm=1000)
    t2 = _matmul_t(x2, Wc1, bm=512)

    g1, g2, s1, c1, s2, c2 = _sc_fused(x0, cluster1, x1, cluster2, t1, t2)

    y0 = _add(x0, g1, bm=1000)
    y1 = _mean_mm_add(x1, g2, s1, c1, Wf0, bm=1000)
    zero2 = jnp.zeros((N2, C), jnp.float32)
    y2 = _mean_mm_add(x2, zero2, s2, c2, Wf1, bm=512)
    return (y0, y1, y2)


# speedup vs baseline: 1.9077x; 1.0019x over previous
import functools

import jax
import jax.numpy as jnp
from jax import lax
from jax.experimental import pallas as pl
from jax.experimental.pallas import tpu as pltpu
from jax.experimental.pallas import tpu_sc as plsc

NC, NS, L = 2, 16, 16
NW = NC * NS

N0, N1, N2, C = 100000, 25000, 6250, 128
G0 = 80
G1R = 40
TRASH = 16


def _cdiv(a, b):
    return (a + b - 1) // b


def _seg_geometry(nseg):
    nseg_pad = _cdiv(nseg, 8) * 8
    H = _cdiv(nseg // 2, 128) * 128
    own = (H, nseg - H)
    wrote = (H, nseg_pad - H)
    accr = _cdiv(max(own) + 8 + TRASH, 8) * 8
    return nseg_pad, H, own, wrote, accr


_PAD1, _H1, _OWN1, _WROTE1, _ACCR1 = _seg_geometry(N1)
_PAD2, _H2, _OWN2, _WROTE2, _ACCR2 = _seg_geometry(N2)
_ACCR = max(_ACCR1, _ACCR2)


def _chunks(total, parts, unit=8):
    u = total // unit
    base, extra = divmod(u, parts)
    out, off = [], 0
    for j in range(parts):
        sz = (base + (1 if j < extra else 0)) * unit
        out.append((off, sz))
        off += sz
    assert off == total
    return out


def _mm_body(x_ref, w_ref, o_ref):
    o_ref[...] = lax.dot_general(
        x_ref[...], w_ref[...], (((1,), (1,)), ((), ())),
        preferred_element_type=jnp.float32)


def _matmul_t(x, w, bm):
    m, c = x.shape
    return pl.pallas_call(
        _mm_body,
        grid=(_cdiv(m, bm),),
        in_specs=[pl.BlockSpec((bm, c), lambda i: (i, 0)),
                  pl.BlockSpec((c, c), lambda i: (0, 0))],
        out_specs=pl.BlockSpec((bm, c), lambda i: (i, 0)),
        out_shape=jax.ShapeDtypeStruct((m, c), jnp.float32),
    )(x, w)


def _add_body(a_ref, b_ref, o_ref):
    o_ref[...] = a_ref[...] + b_ref[...]


def _add(a, b, bm):
    m, c = a.shape
    return pl.pallas_call(
        _add_body,
        grid=(_cdiv(m, bm),),
        in_specs=[pl.BlockSpec((bm, c), lambda i: (i, 0)),
                  pl.BlockSpec((bm, c), lambda i: (i, 0))],
        out_specs=pl.BlockSpec((bm, c), lambda i: (i, 0)),
        out_shape=jax.ShapeDtypeStruct((m, c), jnp.float32),
    )(a, b)


def _mean_mm_body(base_ref, b2_ref, s_ref, c_ref, w_ref, o_ref):
    cnt = jnp.maximum(c_ref[...][:, 0:1], 1.0)
    mean = s_ref[...] / cnt
    o_ref[...] = base_ref[...] + b2_ref[...] + lax.dot_general(
        mean, w_ref[...], (((1,), (1,)), ((), ())),
        preferred_element_type=jnp.float32)


def _mean_mm_add(base, base2, sums, cnts, w, bm):
    m, c = base.shape
    return pl.pallas_call(
        _mean_mm_body,
        grid=(_cdiv(m, bm),),
        in_specs=[pl.BlockSpec((bm, c), lambda i: (i, 0)),
                  pl.BlockSpec((bm, c), lambda i: (i, 0)),
                  pl.BlockSpec((bm, c), lambda i: (i, 0)),
                  pl.BlockSpec((bm, c), lambda i: (i, 0)),
                  pl.BlockSpec((c, c), lambda i: (0, 0))],
        out_specs=pl.BlockSpec((bm, c), lambda i: (i, 0)),
        out_shape=jax.ShapeDtypeStruct((m, c), jnp.float32),
    )(base, base2, sums, cnts, w)


def _sc_fused(x0, cluster1, x1, cluster2, t1, t2):
    zchunks1 = _chunks(_ACCR1, NS)
    zchunks2 = _chunks(_ACCR2, NS)
    zmax = max(sz for _, sz in zchunks1 + zchunks2)
    zeros = jnp.zeros((zmax, C), jnp.float32)
    ones = jnp.ones((G0, C), jnp.float32)

    @functools.partial(
        pl.kernel,
        out_type=(jax.ShapeDtypeStruct((N0, C), jnp.float32),
                  jax.ShapeDtypeStruct((N1, C), jnp.float32),
                  jax.ShapeDtypeStruct((_PAD1, C), jnp.float32),
                  jax.ShapeDtypeStruct((_PAD1, C), jnp.float32),
                  jax.ShapeDtypeStruct((_PAD2, C), jnp.float32),
                  jax.ShapeDtypeStruct((_PAD2, C), jnp.float32)),
        mesh=plsc.VectorSubcoreMesh(core_axis_name="c", subcore_axis_name="s",
                                    num_cores=NC, num_subcores=NS),
        scratch_types=[
            pltpu.VMEM((G0,), jnp.int32),
            pltpu.VMEM((G0,), jnp.int32),
            pltpu.VMEM((G0,), jnp.int32),
            pltpu.VMEM((G0,), jnp.int32),
            pltpu.VMEM((G1R,), jnp.int32),
            pltpu.VMEM((G1R,), jnp.int32),
            pltpu.VMEM((G0, C), jnp.float32),
            pltpu.VMEM((G0, C), jnp.float32),
            pltpu.VMEM_SHARED((_ACCR, C), jnp.float32),
            pltpu.SemaphoreType.DMA,
            pltpu.SemaphoreType.DMA,
            pltpu.SemaphoreType.DMA,
            pltpu.SemaphoreType.DMA,
        ],
    )
    def k(x0_hbm, cl1_hbm, x1_hbm, cl2_hbm, t1_hbm, t2_hbm,
          z_hbm, o_hbm,
          g1_hbm, g2_hbm, s1_hbm, c1_hbm, s2_hbm, c2_hbm,
          iva, ivb, liva, livb, liv40a, liv40b, xva, xvb, sacc,
          semA, semB, semSa, semSb):
        cid = lax.axis_index("c")
        sid = lax.axis_index("s")
        wid = sid * NC + cid

        def pipeline(nblk, first, stride, start, finish, drain):
            cnt = (nblk - 1 - first) // stride + 1

            def blk(i):
                return first + i * stride

            start(blk(0), 0)
            start(blk(1), 1)
            finish(blk(0), 0)

            @pl.when(2 < cnt)
            def _():
                drain(0)
                start(blk(2), 0)

            finish(blk(1), 1)

            @pl.loop(1, cnt // 2)
            def _(t):
                i0 = 2 * t
                drain(1)
                start(blk(i0 + 1), 1)
                finish(blk(i0), 0)

                @pl.when(i0 + 2 < cnt)
                def _():
                    drain(0)
                    start(blk(i0 + 2), 0)

                finish(blk(i0 + 1), 1)

            @pl.when(cnt % 2 == 1)
            def _():
                finish(blk(cnt - 1), 0)

            drain(0)
            drain(1)

        bufs = ((iva, xva, semA, semSa), (ivb, xvb, semB, semSb))

        def make_drain(grp):
            def drain(p):
                iv, xv, sem, semS = bufs[p]
                pltpu.make_async_copy(o_hbm.at[pl.ds(0, grp)],
                                      xv.at[pl.ds(0, grp)], semS).wait()
            return drain

        def gather_phase(cl_hbm, tab_hbm, g_hbm, n, grp):
            def start(b, p):
                iv, xv, sem, semS = bufs[p]
                pltpu.async_copy(cl_hbm.at[pl.ds(b * grp, grp)],
                                 iv.at[pl.ds(0, grp)], sem)

            def finish(b, p):
                iv, xv, sem, semS = bufs[p]
                pltpu.make_async_copy(cl_hbm.at[pl.ds(b * grp, grp)],
                                      iv.at[pl.ds(0, grp)], sem).wait()
                pltpu.async_copy(tab_hbm.at[iv.at[pl.ds(0, grp)]],
                                 xv.at[pl.ds(0, grp)], sem).wait()
                pltpu.async_copy(xv.at[pl.ds(0, grp)],
                                 g_hbm.at[pl.ds(b * grp, grp)], semS)

            pipeline(n // grp, wid, NW, start, finish, make_drain(grp))

        gather_phase(cl1_hbm, t1_hbm, g1_hbm, N0, G0)
        gather_phase(cl2_hbm, t2_hbm, g2_hbm, N1, G1R)

        def zero_acc(zchunks):
            plsc.subcore_barrier()
            for ss, (off, sz) in enumerate(zchunks):
                if sz == 0:
                    continue

                @pl.when(sid == ss)
                def _(off=off, sz=sz):
                    pltpu.sync_copy(z_hbm.at[pl.ds(0, sz)],
                                    sacc.at[pl.ds(off, sz)])
            plsc.subcore_barrier()

        def writeback(out_hbm, wrote, H):
            plsc.subcore_barrier()
            for cc in range(NC):
                for ss, (off, sz) in enumerate(_chunks(wrote[cc], NS)):
                    if sz == 0:
                        continue

                    @pl.when((cid == cc) & (sid == ss))
                    def _(off=off, sz=sz, ob=cc * H):
                        pltpu.sync_copy(sacc.at[pl.ds(off, sz)],
                                        out_hbm.at[pl.ds(ob + off, sz)])

        def segsum_phase(x_hbm, cl_hbm, sums_hbm, cnts_hbm, n,
                         H, own, wrote, grp, livs, zchunks):
            base = cid * H
            nown = jnp.where(cid == 0, own[0], own[1])

            def scatter_sweep(load_x):
                def start(b, p):
                    iv, xv, sem, semS = bufs[p]
                    pltpu.async_copy(cl_hbm.at[pl.ds(b * grp, grp)],
                                     iv.at[pl.ds(0, grp)], sem)
                    if load_x:
                        pltpu.async_copy(x_hbm.at[pl.ds(b * grp, grp)],
                                         xv.at[pl.ds(0, grp)], sem)

                def finish(b, p):
                    iv, xv, sem, semS = bufs[p]
                    lref = livs[p]
                    pltpu.make_async_copy(cl_hbm.at[pl.ds(b * grp, grp)],
                                          iv.at[pl.ds(0, grp)], sem).wait()
                    if load_x:
                        pltpu.make_async_copy(x_hbm.at[pl.ds(b * grp, grp)],
                                              xv.at[pl.ds(0, grp)],
                                              sem).wait()

                    @pl.loop(0, grp, step=L)
                    def _(j):
                        v = iv[pl.ds(j, L)] - base
                        inb = (v >= 0) & (v < nown)
                        trash = (nown + 8 +
                                 ((lax.iota(jnp.int32, L) + j) & (TRASH - 1)))
                        lref[pl.ds(j, L)] = jnp.where(inb, v, trash)

                    pltpu.async_copy(xv.at[pl.ds(0, grp)],
                                     sacc.at[lref], semS, add=True)

                pipeline(n // grp, sid, NS, start, finish, make_drain(grp))

            zero_acc(zchunks)
            scatter_sweep(True)
            writeback(sums_hbm, wrote, H)

            zero_acc(zchunks)
            pltpu.sync_copy(o_hbm.at[pl.ds(0, G0)], xva)
            pltpu.sync_copy(o_hbm.at[pl.ds(0, G0)], xvb)
            scatter_sweep(False)
            writeback(cnts_hbm, wrote, H)

        segsum_phase(x0_hbm, cl1_hbm, s1_hbm, c1_hbm, N0,
                     _H1, _OWN1, _WROTE1, G0, (liva, livb), zchunks1)
        segsum_phase(x1_hbm, cl2_hbm, s2_hbm, c2_hbm, N1,
                     _H2, _OWN2, _WROTE2, G1R, (liv40a, liv40b), zchunks2)

    return k(x0, cluster1, x1, cluster2, t1, t2, zeros, ones)


def kernel(x0, x1, x2, cluster1, cluster2, Wf0, Wf1, Wc0, Wc1):
    t1 = _matmul_t(x1, Wc0, bm=1000)
    t2 = _matmul_t(x2, Wc1, bm=512)

    g1, g2, s1, c1, s2, c2 = _sc_fused(x0, cluster1, x1, cluster2, t1, t2)

    y0 = _add(x0, g1, bm=1000)
    y1 = _mean_mm_add(x1, g2, s1, c1, Wf0, bm=1000)
    zero2 = jnp.zeros((N2, C), jnp.float32)
    y2 = _mean_mm_add(x2, zero2, s2, c2, Wf1, bm=512)
    return (y0, y1, y2)
